# Initial kernel scaffold; baseline (speedup 1.0000x reference)
import jax
import jax.numpy as jnp
from jax.experimental import pallas as pl


def kernel(text, max_seq_len, embed_table):
    raise NotImplementedError("write your pallas kernel here")


# speedup vs baseline: 3.0300x; 3.0300x over previous
import functools

import jax
import jax.numpy as jnp
from jax import lax
from jax.experimental import pallas as pl
from jax.experimental.pallas import tpu as pltpu
from jax.experimental.pallas import tpu_sc as plsc

_NC = 2
_NS = 16
_NW = _NC * _NS
_LANES = 16

_CHUNK = 1024
_IDX_ROWS = _CHUNK // 128


def _make_embed(num_tok, seq_len, vocab, dim):
    assert num_tok % (_NW * _CHUNK) == 0
    b_per_w = num_tok // _NW
    n_chunks = b_per_w // _CHUNK
    mesh = plsc.VectorSubcoreMesh(core_axis_name="c", subcore_axis_name="s")

    @functools.partial(
        pl.kernel,
        mesh=mesh,
        compiler_params=pltpu.CompilerParams(use_tc_tiling_on_sc=False),
        out_type=jax.ShapeDtypeStruct((num_tok, dim), jnp.float32),
        scratch_types=[
            pltpu.VMEM((_CHUNK,), jnp.int32),
            pltpu.VMEM((_IDX_ROWS, 128), jnp.int32),
            pltpu.VMEM((_CHUNK, dim), jnp.float32),
            pltpu.VMEM((_LANES,), jnp.int32),
            pltpu.SemaphoreType.DMA,
        ],
    )
    def embed(text_hbm, msl_hbm, table_hbm, out_hbm, tok_v, idx_v, rows_v,
              msl_v, sem):
        wid = lax.axis_index("s") * _NC + lax.axis_index("c")
        gbase = wid * b_per_w
        pltpu.sync_copy(msl_hbm, msl_v)
        msl_vec = msl_v[...]
        lane = lax.iota(jnp.int32, _LANES)

        def chunk_body(j, carry):
            gofs = gbase + j * _CHUNK
            pltpu.sync_copy(text_hbm.at[pl.ds(gofs, _CHUNK)], tok_v)
            for r in range(_IDX_ROWS):
                for i in range(128 // _LANES):
                    o = r * 128 + i * _LANES
                    col = lax.rem(lane + (j * _CHUNK + o), seq_len)
                    tok = tok_v[pl.ds(o, _LANES)]
                    idx_v[r, pl.ds(i * _LANES, _LANES)] = jnp.where(
                        col < msl_vec, tok + 1, 0)
            copies = [
                pltpu.async_copy(
                    table_hbm.at[idx_v.at[r]],
                    rows_v.at[pl.ds(r * 128, 128)],
                    sem,
                )
                for r in range(_IDX_ROWS)
            ]
            for c in copies:
                c.wait()
            pltpu.sync_copy(rows_v, out_hbm.at[pl.ds(gofs, _CHUNK)])
            return carry

        lax.fori_loop(0, n_chunks, chunk_body, 0)

    return embed


def kernel(text, max_seq_len, embed_table):
    bsz, seq_len = text.shape
    vocab, dim = embed_table.shape
    num_tok = bsz * seq_len
    text_flat = text.reshape(num_tok)
    msl = jnp.full((_LANES,), max_seq_len, dtype=jnp.int32)
    out = _make_embed(num_tok, seq_len, vocab, dim)(
        text_flat, msl, embed_table)
    return out.reshape(bsz, seq_len, dim)


# speedup vs baseline: 3.2519x; 1.0732x over previous
import functools

import jax
import jax.numpy as jnp
from jax import lax
from jax.experimental import pallas as pl
from jax.experimental.pallas import tpu as pltpu
from jax.experimental.pallas import tpu_sc as plsc

_NC = 2
_NS = 16
_NW = _NC * _NS
_LANES = 16

_CHUNK = 640
_IDX_ROWS = _CHUNK // 128
_PDIM = 128


def _make_embed(num_tok, seq_len, vocab):
    assert num_tok % (_NW * _CHUNK) == 0
    b_per_w = num_tok // _NW
    n_chunks = b_per_w // _CHUNK
    mesh = plsc.VectorSubcoreMesh(core_axis_name="c", subcore_axis_name="s")

    @functools.partial(
        pl.kernel,
        mesh=mesh,
        out_type=jax.ShapeDtypeStruct((num_tok, _PDIM), jnp.float32),
        scratch_types=[
            pltpu.VMEM((_CHUNK,), jnp.int32),
            pltpu.VMEM((_IDX_ROWS, 128), jnp.int32),
            pltpu.VMEM((_CHUNK, _PDIM), jnp.float32),
            pltpu.VMEM((_LANES,), jnp.int32),
            pltpu.SemaphoreType.DMA,
        ],
    )
    def embed(text_hbm, msl_hbm, table_hbm, out_hbm, tok_v, idx_v, rows_v,
              msl_v, sem):
        wid = lax.axis_index("s") * _NC + lax.axis_index("c")
        gbase = wid * b_per_w
        pltpu.sync_copy(msl_hbm, msl_v)
        msl_vec = msl_v[...]
        lane = lax.iota(jnp.int32, _LANES)

        def chunk_body(j, carry):
            gofs = gbase + j * _CHUNK
            pltpu.sync_copy(text_hbm.at[pl.ds(gofs, _CHUNK)], tok_v)
            for r in range(_IDX_ROWS):
                for i in range(128 // _LANES):
                    o = r * 128 + i * _LANES
                    col = lax.rem(lane + (j * _CHUNK + o), seq_len)
                    tok = tok_v[pl.ds(o, _LANES)]
                    idx_v[r, pl.ds(i * _LANES, _LANES)] = jnp.where(
                        col < msl_vec, tok + 1, 0)
            copies = [
                pltpu.async_copy(
                    table_hbm.at[idx_v.at[r]],
                    rows_v.at[pl.ds(r * 128, 128)],
                    sem,
                )
                for r in range(_IDX_ROWS)
            ]
            for c in copies:
                c.wait()
            pltpu.sync_copy(rows_v, out_hbm.at[pl.ds(gofs, _CHUNK)])
            return carry

        lax.fori_loop(0, n_chunks, chunk_body, 0)

    return embed


def kernel(text, max_seq_len, embed_table):
    bsz, seq_len = text.shape
    vocab, dim = embed_table.shape
    num_tok = bsz * seq_len
    text_flat = text.reshape(num_tok)
    table_pad = jnp.pad(embed_table, ((0, 0), (0, _PDIM - dim)))
    msl = jnp.full((_LANES,), max_seq_len, dtype=jnp.int32)
    out = _make_embed(num_tok, seq_len, vocab)(text_flat, msl, table_pad)
    return out[:, :dim].reshape(bsz, seq_len, dim)
